# baseline (speedup 1.0000x reference)
import functools

import jax
import jax.numpy as jnp
import numpy as np
from jax import lax
from jax.experimental import pallas as pl
from jax.experimental.pallas import tpu as pltpu
from jax.experimental.pallas import tpu_sc as plsc

NUM_CORES = 2
NUM_SUBCORES = 16
NUM_WORKERS = NUM_CORES * NUM_SUBCORES
CHUNK = 128
NBUF = 4
ROW_UNROLL = 4
HI_MASK = -65536


@functools.partial(jax.jit, static_argnums=(2, 3))
def _embedding_gather(x_flat, W_packed, B, D):
  b_per_w = B // NUM_WORKERS
  n_chunks = b_per_w // CHUNK
  n_groups = n_chunks // NBUF
  Dw = D // 2
  mesh = plsc.VectorSubcoreMesh(
      core_axis_name="c", subcore_axis_name="s",
      num_cores=NUM_CORES, num_subcores=NUM_SUBCORES)

  @functools.partial(
      pl.kernel,
      out_type=jax.ShapeDtypeStruct((B, D), jnp.float32),
      mesh=mesh,
      compiler_params=pltpu.CompilerParams(
          needs_layout_passes=False, use_tc_tiling_on_sc=False),
      scratch_types=(
          [pltpu.VMEM((b_per_w,), jnp.int32)]
          + [pltpu.VMEM((CHUNK, Dw), jnp.int32) for _ in range(NBUF)]
          + [pltpu.VMEM((CHUNK, D), jnp.float32) for _ in range(NBUF)]
          + [pltpu.SemaphoreType.DMA for _ in range(2 * NBUF)]
      ),
  )
  def k(table_hbm, idx_hbm, out_hbm, idx_all, *bufs_and_sems):
    rows16 = bufs_and_sems[:NBUF]
    rows32 = bufs_and_sems[NBUF:2 * NBUF]
    sg = bufs_and_sems[2 * NBUF:3 * NBUF]
    sw = bufs_and_sems[3 * NBUF:4 * NBUF]
    wid = lax.axis_index("s") * NUM_CORES + lax.axis_index("c")
    base = wid * b_per_w

    pltpu.sync_copy(idx_hbm.at[pl.ds(base, b_per_w)], idx_all)

    def start_gather(i, b):
      pltpu.async_copy(
          table_hbm.at[idx_all.at[pl.ds(i * CHUNK, CHUNK)]], rows16[b], sg[b])

    def wait_gather(b):
      pltpu.make_async_copy(
          table_hbm.at[idx_all.at[pl.ds(0, CHUNK)]], rows16[b], sg[b]).wait()

    def start_wb(i, b):
      pltpu.async_copy(rows32[b], out_hbm.at[pl.ds(base + i * CHUNK, CHUNK)],
                       sw[b])

    def wait_wb(b):
      pltpu.make_async_copy(rows32[b], out_hbm.at[pl.ds(base, CHUNK)],
                            sw[b]).wait()

    lane = lax.iota(jnp.int32, 16)

    def widen_chunk(b):
      def row_body(r0, carry):
        for rr in range(ROW_UNROLL):
          r = r0 * ROW_UNROLL + rr
          row_vec = jnp.broadcast_to(r, (16,)).astype(jnp.int32)
          for g in range(Dw // 16):
            v = rows16[b][r, pl.ds(g * 16, 16)]
            lo = plsc.bitcast(lax.shift_left(v, 16), jnp.float32)
            hi = plsc.bitcast(lax.bitwise_and(v, np.int32(HI_MASK)),
                              jnp.float32)
            col_even = 32 * g + 2 * lane
            plsc.store_scatter(rows32[b], [row_vec, col_even], lo)
            plsc.store_scatter(rows32[b], [row_vec, col_even + 1], hi)
        return carry

      lax.fori_loop(0, CHUNK // ROW_UNROLL, row_body, 0)

    for b in range(NBUF):
      start_gather(b, b)

    def group(g, carry):
      for b in range(NBUF):
        wait_gather(b)

        @pl.when(g > 0)
        def _():
          wait_wb(b)

        widen_chunk(b)
        start_wb(g * NBUF + b, b)

        @pl.when(g + 1 < n_groups)
        def _():
          start_gather((g + 1) * NBUF + b, b)
      return carry

    lax.fori_loop(0, n_groups, group, 0)
    for b in range(NBUF):
      wait_wb(b)

  return k(W_packed, x_flat)


def kernel(x, W):
  batch, seq = x.shape
  D = W.shape[-1]
  B = batch * seq
  x_flat = x.reshape(B).astype(jnp.int32)
  W16 = W.astype(jnp.bfloat16)
  W_packed = jax.lax.bitcast_convert_type(
      W16.reshape(W.shape[0], D // 2, 2), jnp.int32)
  out = _embedding_gather(x_flat, W_packed, B, D)
  return out.reshape(batch, seq, D)


# speedup vs baseline: 1.5684x; 1.5684x over previous
import functools

import jax
import jax.numpy as jnp
import numpy as np
from jax import lax
from jax.experimental import pallas as pl
from jax.experimental.pallas import tpu as pltpu
from jax.experimental.pallas import tpu_sc as plsc

NUM_CORES = 2
NUM_SUBCORES = 16
NUM_WORKERS = NUM_CORES * NUM_SUBCORES
CHUNK = 128
NBUF = 4
ROW_UNROLL = 4
HI_MASK = -65536


@functools.partial(jax.jit, static_argnums=(2, 3))
def _embedding_gather(x_flat, W_packed, B, D):
  b_per_w = B // NUM_WORKERS
  n_chunks = b_per_w // CHUNK
  n_groups = n_chunks // NBUF
  Dw = D // 2
  mesh = plsc.VectorSubcoreMesh(
      core_axis_name="c", subcore_axis_name="s",
      num_cores=NUM_CORES, num_subcores=NUM_SUBCORES)

  @functools.partial(
      pl.kernel,
      out_type=jax.ShapeDtypeStruct((B, D), jnp.float32),
      mesh=mesh,
      compiler_params=pltpu.CompilerParams(
          needs_layout_passes=False, use_tc_tiling_on_sc=False),
      scratch_types=(
          [pltpu.VMEM((b_per_w,), jnp.int32)]
          + [pltpu.VMEM((CHUNK, Dw), jnp.int32) for _ in range(NBUF)]
          + [pltpu.VMEM((CHUNK, D), jnp.float32) for _ in range(NBUF)]
          + [pltpu.SemaphoreType.DMA for _ in range(2 * NBUF)]
      ),
  )
  def k(table_hbm, idx_hbm, out_hbm, idx_all, *bufs_and_sems):
    rows16 = bufs_and_sems[:NBUF]
    rows32 = bufs_and_sems[NBUF:2 * NBUF]
    sg = bufs_and_sems[2 * NBUF:3 * NBUF]
    sw = bufs_and_sems[3 * NBUF:4 * NBUF]
    wid = lax.axis_index("s") * NUM_CORES + lax.axis_index("c")
    base = wid * b_per_w

    pltpu.sync_copy(idx_hbm.at[pl.ds(base, b_per_w)], idx_all)

    def start_gather(i, b):
      pltpu.async_copy(
          table_hbm.at[idx_all.at[pl.ds(i * CHUNK, CHUNK)]], rows16[b], sg[b])

    def wait_gather(b):
      pltpu.make_async_copy(
          table_hbm.at[idx_all.at[pl.ds(0, CHUNK)]], rows16[b], sg[b]).wait()

    def start_wb(i, b):
      pltpu.async_copy(rows32[b], out_hbm.at[pl.ds(base + i * CHUNK, CHUNK)],
                       sw[b])

    def wait_wb(b):
      pltpu.make_async_copy(rows32[b], out_hbm.at[pl.ds(base, CHUNK)],
                            sw[b]).wait()

    def widen_chunk(b):
      def row_body(r0, carry):
        for rr in range(ROW_UNROLL):
          r = r0 * ROW_UNROLL + rr
          for g in range(Dw // 16):
            v = rows16[b][r, pl.ds(g * 16, 16)]
            rows32[b][r, pl.ds(g * 16, 16)] = plsc.bitcast(
                lax.shift_left(v, 16), jnp.float32)
            rows32[b][r, pl.ds(Dw + g * 16, 16)] = plsc.bitcast(
                lax.bitwise_and(v, np.int32(HI_MASK)), jnp.float32)
        return carry

      lax.fori_loop(0, CHUNK // ROW_UNROLL, row_body, 0)

    for b in range(NBUF):
      start_gather(b, b)

    def group(g, carry):
      for b in range(NBUF):
        wait_gather(b)

        @pl.when(g > 0)
        def _():
          wait_wb(b)

        widen_chunk(b)
        start_wb(g * NBUF + b, b)

        @pl.when(g + 1 < n_groups)
        def _():
          start_gather((g + 1) * NBUF + b, b)
      return carry

    lax.fori_loop(0, n_groups, group, 0)
    for b in range(NBUF):
      wait_wb(b)

  return k(W_packed, x_flat)


def kernel(x, W):
  batch, seq = x.shape
  D = W.shape[-1]
  B = batch * seq
  x_flat = x.reshape(B).astype(jnp.int32)
  bits = jax.lax.bitcast_convert_type(W, jnp.uint32)
  lo, hi = bits[:, : D // 2], bits[:, D // 2:]
  rnd = lambda u: u + jnp.uint32(0x7FFF) + ((u >> 16) & jnp.uint32(1))
  W_packed = jax.lax.bitcast_convert_type(
      (rnd(lo) >> 16) | (rnd(hi) & jnp.uint32(0xFFFF0000)), jnp.int32)
  out = _embedding_gather(x_flat, W_packed, B, D)
  return out.reshape(batch, seq, D)


# speedup vs baseline: 2.5672x; 1.6369x over previous
import functools

import jax
import jax.numpy as jnp
import numpy as np
from jax import lax
from jax.experimental import pallas as pl
from jax.experimental.pallas import tpu as pltpu
from jax.experimental.pallas import tpu_sc as plsc

NUM_CORES = 2
NUM_SUBCORES = 16
NUM_WORKERS = NUM_CORES * NUM_SUBCORES
CHUNK = 128
NBUF = 4
ROW_UNROLL = 4
HI_MASK = -65536


@functools.partial(jax.jit, static_argnums=(2, 3))
def _embedding_gather(x_flat, W_packed, B, D):
  b_per_w = B // NUM_WORKERS
  n_chunks = b_per_w // CHUNK
  n_groups = n_chunks // NBUF
  Dw = D // 2
  mesh = plsc.VectorSubcoreMesh(
      core_axis_name="c", subcore_axis_name="s",
      num_cores=NUM_CORES, num_subcores=NUM_SUBCORES)

  @functools.partial(
      pl.kernel,
      out_type=jax.ShapeDtypeStruct((B, D), jnp.float32),
      mesh=mesh,
      compiler_params=pltpu.CompilerParams(
          needs_layout_passes=False, use_tc_tiling_on_sc=False),
      scratch_types=(
          [pltpu.VMEM((b_per_w,), jnp.int32)]
          + [pltpu.VMEM((CHUNK, Dw), jnp.int32) for _ in range(NBUF)]
          + [pltpu.VMEM((CHUNK, D), jnp.float32) for _ in range(NBUF)]
          + [pltpu.SemaphoreType.DMA for _ in range(2 * NBUF)]
      ),
  )
  def k(table_hbm, idx_hbm, out_hbm, idx_all, *bufs_and_sems):
    rows16 = bufs_and_sems[:NBUF]
    rows32 = bufs_and_sems[NBUF:2 * NBUF]
    sg = bufs_and_sems[2 * NBUF:3 * NBUF]
    sw = bufs_and_sems[3 * NBUF:4 * NBUF]
    wid = lax.axis_index("s") * NUM_CORES + lax.axis_index("c")
    base = wid * b_per_w

    pltpu.sync_copy(idx_hbm.at[pl.ds(base, b_per_w)], idx_all)

    def start_gather(i, b):
      pltpu.async_copy(
          table_hbm.at[idx_all.at[pl.ds(i * CHUNK, CHUNK)]], rows16[b], sg[b])

    def wait_gather(b):
      pltpu.make_async_copy(
          table_hbm.at[idx_all.at[pl.ds(0, CHUNK)]], rows16[b], sg[b]).wait()

    def start_wb(i, b):
      pltpu.async_copy(rows32[b], out_hbm.at[pl.ds(base + i * CHUNK, CHUNK)],
                       sw[b])

    def wait_wb(b):
      pltpu.make_async_copy(rows32[b], out_hbm.at[pl.ds(base, CHUNK)],
                            sw[b]).wait()

    def widen_chunk(b):
      @plsc.parallel_loop(0, CHUNK, step=ROW_UNROLL, unroll=2)
      def row_body(r0):
        for rr in range(ROW_UNROLL):
          r = r0 + rr
          for g in range(Dw // 16):
            v = rows16[b][r, pl.ds(g * 16, 16)]
            rows32[b][r, pl.ds(g * 16, 16)] = plsc.bitcast(
                lax.shift_left(v, 16), jnp.float32)
            rows32[b][r, pl.ds(Dw + g * 16, 16)] = plsc.bitcast(
                lax.bitwise_and(v, np.int32(HI_MASK)), jnp.float32)

    for b in range(NBUF):
      start_gather(b, b)

    def group(g, carry):
      for b in range(NBUF):
        wait_gather(b)

        @pl.when(g > 0)
        def _():
          wait_wb(b)

        widen_chunk(b)
        start_wb(g * NBUF + b, b)

        @pl.when(g + 1 < n_groups)
        def _():
          start_gather((g + 1) * NBUF + b, b)
      return carry

    lax.fori_loop(0, n_groups, group, 0)
    for b in range(NBUF):
      wait_wb(b)

  return k(W_packed, x_flat)


def kernel(x, W):
  batch, seq = x.shape
  D = W.shape[-1]
  B = batch * seq
  x_flat = x.reshape(B).astype(jnp.int32)
  bits = jax.lax.bitcast_convert_type(W, jnp.uint32)
  lo, hi = bits[:, : D // 2], bits[:, D // 2:]
  rnd = lambda u: u + jnp.uint32(0x7FFF) + ((u >> 16) & jnp.uint32(1))
  W_packed = jax.lax.bitcast_convert_type(
      (rnd(lo) >> 16) | (rnd(hi) & jnp.uint32(0xFFFF0000)), jnp.int32)
  out = _embedding_gather(x_flat, W_packed, B, D)
  return out.reshape(batch, seq, D)


# speedup vs baseline: 2.5737x; 1.0025x over previous
import functools

import jax
import jax.numpy as jnp
import numpy as np
from jax import lax
from jax.experimental import pallas as pl
from jax.experimental.pallas import tpu as pltpu
from jax.experimental.pallas import tpu_sc as plsc

NUM_CORES = 2
NUM_SUBCORES = 16
NUM_WORKERS = NUM_CORES * NUM_SUBCORES
CHUNK = 128
NBUF = 4
ROW_UNROLL = 4
HI_MASK = -65536


@functools.partial(jax.jit, static_argnums=(2, 3))
def _embedding_gather(x_flat, W_packed, B, D):
  b_per_w = B // NUM_WORKERS
  n_chunks = b_per_w // CHUNK
  n_groups = n_chunks // NBUF
  Dw = D // 2
  mesh = plsc.VectorSubcoreMesh(
      core_axis_name="c", subcore_axis_name="s",
      num_cores=NUM_CORES, num_subcores=NUM_SUBCORES)

  @functools.partial(
      pl.kernel,
      out_type=jax.ShapeDtypeStruct((B, D), jnp.float32),
      mesh=mesh,
      compiler_params=pltpu.CompilerParams(
          needs_layout_passes=False, use_tc_tiling_on_sc=False),
      scratch_types=(
          [pltpu.VMEM((b_per_w,), jnp.int32)]
          + [pltpu.VMEM((CHUNK, Dw), jnp.int32) for _ in range(NBUF)]
          + [pltpu.VMEM((CHUNK, D), jnp.float32) for _ in range(NBUF)]
          + [pltpu.SemaphoreType.DMA for _ in range(2 * NBUF)]
      ),
  )
  def k(table_hbm, idx_hbm, out_hbm, idx_all, *bufs_and_sems):
    rows16 = bufs_and_sems[:NBUF]
    rows32 = bufs_and_sems[NBUF:2 * NBUF]
    sg = bufs_and_sems[2 * NBUF:3 * NBUF]
    sw = bufs_and_sems[3 * NBUF:4 * NBUF]
    wid = lax.axis_index("s") * NUM_CORES + lax.axis_index("c")
    base = wid * b_per_w

    pltpu.sync_copy(idx_hbm.at[pl.ds(base, b_per_w)], idx_all)

    def start_gather(i, b):
      pltpu.async_copy(
          table_hbm.at[idx_all.at[pl.ds(i * CHUNK, CHUNK)]], rows16[b], sg[b])

    def wait_gather(b):
      pltpu.make_async_copy(
          table_hbm.at[idx_all.at[pl.ds(0, CHUNK)]], rows16[b], sg[b]).wait()

    def start_wb(i, b):
      pltpu.async_copy(rows32[b], out_hbm.at[pl.ds(base + i * CHUNK, CHUNK)],
                       sw[b])

    def wait_wb(b):
      pltpu.make_async_copy(rows32[b], out_hbm.at[pl.ds(base, CHUNK)],
                            sw[b]).wait()

    def widen_chunk(b):
      @plsc.parallel_loop(0, CHUNK, step=ROW_UNROLL, unroll=4)
      def row_body(r0):
        for rr in range(ROW_UNROLL):
          r = r0 + rr
          for g in range(Dw // 16):
            v = rows16[b][r, pl.ds(g * 16, 16)]
            rows32[b][r, pl.ds(g * 16, 16)] = plsc.bitcast(
                lax.shift_left(v, 16), jnp.float32)
            rows32[b][r, pl.ds(Dw + g * 16, 16)] = plsc.bitcast(
                lax.bitwise_and(v, np.int32(HI_MASK)), jnp.float32)

    for b in range(NBUF):
      start_gather(b, b)

    def group(g, carry):
      for b in range(NBUF):
        wait_gather(b)

        @pl.when(g > 0)
        def _():
          wait_wb(b)

        widen_chunk(b)
        start_wb(g * NBUF + b, b)

        @pl.when(g + 1 < n_groups)
        def _():
          start_gather((g + 1) * NBUF + b, b)
      return carry

    lax.fori_loop(0, n_groups, group, 0)
    for b in range(NBUF):
      wait_wb(b)

  return k(W_packed, x_flat)


def kernel(x, W):
  batch, seq = x.shape
  D = W.shape[-1]
  B = batch * seq
  x_flat = x.reshape(B).astype(jnp.int32)
  bits = jax.lax.bitcast_convert_type(W, jnp.uint32)
  lo, hi = bits[:, : D // 2], bits[:, D // 2:]
  rnd = lambda u: u + jnp.uint32(0x7FFF) + ((u >> 16) & jnp.uint32(1))
  W_packed = jax.lax.bitcast_convert_type(
      (rnd(lo) >> 16) | (rnd(hi) & jnp.uint32(0xFFFF0000)), jnp.int32)
  out = _embedding_gather(x_flat, W_packed, B, D)
  return out.reshape(batch, seq, D)


# speedup vs baseline: 3.0194x; 1.1732x over previous
import functools

import jax
import jax.numpy as jnp
from jax import lax
from jax.experimental import pallas as pl
from jax.experimental.pallas import tpu as pltpu
from jax.experimental.pallas import tpu_sc as plsc

NUM_CORES = 2
NUM_SUBCORES = 16
NUM_WORKERS = NUM_CORES * NUM_SUBCORES
CHUNK = 128
NBUF = 4


@functools.partial(jax.jit, static_argnums=(2, 3))
def _embedding_gather(x_flat, W, B, D):
  b_per_w = B // NUM_WORKERS
  n_chunks = b_per_w // CHUNK
  n_groups = n_chunks // NBUF
  mesh = plsc.VectorSubcoreMesh(
      core_axis_name="c", subcore_axis_name="s",
      num_cores=NUM_CORES, num_subcores=NUM_SUBCORES)

  @functools.partial(
      pl.kernel,
      out_type=jax.ShapeDtypeStruct((B, D), jnp.float32),
      mesh=mesh,
      scratch_types=(
          [pltpu.VMEM((b_per_w,), jnp.int32)]
          + [pltpu.VMEM((CHUNK, D), jnp.float32) for _ in range(NBUF)]
          + [pltpu.SemaphoreType.DMA for _ in range(2 * NBUF)]
      ),
  )
  def k(table_hbm, idx_hbm, out_hbm, idx_all, *bufs_and_sems):
    rows = bufs_and_sems[:NBUF]
    sg = bufs_and_sems[NBUF:2 * NBUF]
    sw = bufs_and_sems[2 * NBUF:3 * NBUF]
    wid = lax.axis_index("s") * NUM_CORES + lax.axis_index("c")
    base = wid * b_per_w

    pltpu.sync_copy(idx_hbm.at[pl.ds(base, b_per_w)], idx_all)

    def start_gather(i, b):
      pltpu.async_copy(
          table_hbm.at[idx_all.at[pl.ds(i * CHUNK, CHUNK)]], rows[b], sg[b])

    def wait_gather(b):
      pltpu.make_async_copy(
          table_hbm.at[idx_all.at[pl.ds(0, CHUNK)]], rows[b], sg[b]).wait()

    def start_wb(i, b):
      pltpu.async_copy(rows[b], out_hbm.at[pl.ds(base + i * CHUNK, CHUNK)],
                       sw[b])

    def wait_wb(b):
      pltpu.make_async_copy(rows[b], out_hbm.at[pl.ds(base, CHUNK)],
                            sw[b]).wait()

    for b in range(NBUF):
      start_gather(b, b)

    def group(g, carry):
      for b in range(NBUF):
        wait_gather(b)
        start_wb(g * NBUF + b, b)
      for b in range(NBUF):
        @pl.when(g + 1 < n_groups)
        def _():
          wait_wb(b)
          start_gather((g + 1) * NBUF + b, b)
      return carry

    lax.fori_loop(0, n_groups, group, 0)
    for b in range(NBUF):
      wait_wb(b)

  return k(W, x_flat)


def kernel(x, W):
  batch, seq = x.shape
  D = W.shape[-1]
  B = batch * seq
  x_flat = x.reshape(B).astype(jnp.int32)
  out = _embedding_gather(x_flat, W, B, D)
  return out.reshape(batch, seq, D)
